# baseline (speedup 1.0000x reference)
import functools

import jax
import jax.numpy as jnp
from jax import lax
from jax.experimental import pallas as pl
from jax.experimental.pallas import tpu as pltpu
from jax.experimental.pallas import tpu_sc as plsc

_K = 8
_RB = 16
_CW = 12800
_NCH = 8
_CW2 = 3200
_BIG = 2**31 - 1
_NEG = float("-inf")


def _ins2(x, c, w1, c1, w2, c2):
    gt1 = x > w1
    gt2 = x > w2
    nw2 = jnp.where(gt1, w1, jnp.where(gt2, x, w2))
    nc2 = jnp.where(gt1, c1, jnp.where(gt2, c, c2))
    nw1 = jnp.where(gt1, x, w1)
    nc1 = jnp.where(gt1, c, c1)
    return nw1, nc1, nw2, nc2


def _fast_body(x_ref, tok_ref, val_ref, bad_ref):
    x = x_ref[...]
    r, v = x.shape

    w1 = x[:, :_CW]
    c1 = jnp.zeros((r, _CW), jnp.int32)
    w2 = jnp.full((r, _CW), _NEG, jnp.float32)
    c2 = jnp.zeros((r, _CW), jnp.int32)
    for c in range(1, _NCH):
        off = c * _CW
        wdt = min(_CW, v - off)
        xc = x[:, off:off + wdt]
        if wdt < _CW:
            xc = jnp.concatenate(
                [xc, jnp.full((r, _CW - wdt), _NEG, jnp.float32)], axis=1)
        w1, c1, w2, c2 = _ins2(xc, jnp.int32(c), w1, c1, w2, c2)

    m0 = jnp.max(w1, axis=-1, keepdims=True)
    se = jnp.sum(jnp.exp(x - m0), axis=-1, keepdims=True)
    lse = m0 + jnp.log(se)

    pos = lax.broadcasted_iota(jnp.int32, (r, _CW), 1)
    i1 = c1 * _CW + pos
    i2 = c2 * _CW + pos

    v1 = w1[:, :_CW2]
    j1 = i1[:, :_CW2]
    v2 = jnp.full((r, _CW2), _NEG, jnp.float32)
    j2 = jnp.zeros((r, _CW2), jnp.int32)
    lvl2 = [(w1, i1), (w2, i2)]
    for (wa, ia) in lvl2:
        for c in range(_CW // _CW2):
            if wa is w1 and c == 0:
                continue
            vc = wa[:, c * _CW2:(c + 1) * _CW2]
            jc = ia[:, c * _CW2:(c + 1) * _CW2]
            gt1 = vc > v1
            gt2 = vc > v2
            v2 = jnp.where(gt1, v1, jnp.where(gt2, vc, v2))
            j2 = jnp.where(gt1, j1, jnp.where(gt2, jc, j2))
            v1 = jnp.where(gt1, vc, v1)
            j1 = jnp.where(gt1, jc, j1)

    cand = jnp.concatenate([v1, v2], axis=1)
    cidx = jnp.concatenate([j1, j2], axis=1)

    toks, vals = [], []
    for j in range(_K):
        mj = jnp.max(cand, axis=-1, keepdims=True)
        ij = jnp.min(jnp.where(cand == mj, cidx, _BIG),
                     axis=-1, keepdims=True)
        toks.append(ij)
        vals.append(mj)
        if j + 1 < _K:
            cand = jnp.where(cidx == ij, _NEG, cand)

    v7 = vals[_K - 1]
    n_ge = jnp.sum((x >= v7).astype(jnp.int32), axis=-1, keepdims=True)
    nbad = jnp.sum((n_ge != _K).astype(jnp.int32))

    tok_ref[...] = jnp.concatenate(toks, axis=1)
    val_ref[...] = jnp.concatenate(vals, axis=1) - lse
    bad_ref[...] = jnp.full(bad_ref.shape, nbad, jnp.int32)


def _exact_body(x_ref, tok_ref, val_ref):
    x = x_ref[...]
    r, v = x.shape
    iota = lax.broadcasted_iota(jnp.int32, (r, v), 1)
    m0 = jnp.max(x, axis=-1, keepdims=True)
    lse = m0 + jnp.log(jnp.sum(jnp.exp(x - m0), axis=-1, keepdims=True))
    work = x
    toks, vals = [], []
    for j in range(_K):
        mj = m0 if j == 0 else jnp.max(work, axis=-1, keepdims=True)
        ij = jnp.min(jnp.where(work == mj, iota, _BIG),
                     axis=-1, keepdims=True)
        toks.append(ij)
        vals.append(mj)
        if j + 1 < _K:
            work = jnp.where(iota == ij, _NEG, work)
    tok_ref[...] = jnp.concatenate(toks, axis=1)
    val_ref[...] = jnp.concatenate(vals, axis=1) - lse


def _topk_fast(logits):
    b, v = logits.shape
    return pl.pallas_call(
        _fast_body,
        grid=(b // _RB,),
        in_specs=[pl.BlockSpec((_RB, v), lambda i: (i, 0))],
        out_specs=[pl.BlockSpec((_RB, _K), lambda i: (i, 0)),
                   pl.BlockSpec((_RB, _K), lambda i: (i, 0)),
                   pl.BlockSpec((_RB, _K), lambda i: (i, 0))],
        out_shape=[jax.ShapeDtypeStruct((b, _K), jnp.int32),
                   jax.ShapeDtypeStruct((b, _K), jnp.float32),
                   jax.ShapeDtypeStruct((b, _K), jnp.int32)],
        compiler_params=pltpu.CompilerParams(
            dimension_semantics=("arbitrary",)),
    )(logits)


def _topk_exact(logits):
    b, v = logits.shape
    rb = 8
    return pl.pallas_call(
        _exact_body,
        grid=(b // rb,),
        in_specs=[pl.BlockSpec((rb, v), lambda i: (i, 0))],
        out_specs=[pl.BlockSpec((rb, _K), lambda i: (i, 0)),
                   pl.BlockSpec((rb, _K), lambda i: (i, 0))],
        out_shape=[jax.ShapeDtypeStruct((b, _K), jnp.int32),
                   jax.ShapeDtypeStruct((b, _K), jnp.float32)],
    )(logits)


def _d2t_adjust(d2t, tokens):
    info = plsc.get_sparse_core_info()
    nc, ns = info.num_cores, info.num_subcores
    nw = nc * ns
    b = tokens.shape[0]
    bpw = b // nw
    mesh = plsc.VectorSubcoreMesh(core_axis_name="c", subcore_axis_name="s")

    @functools.partial(
        pl.kernel, mesh=mesh,
        out_type=jax.ShapeDtypeStruct((b,), jnp.int32),
        scratch_types=[pltpu.VMEM((bpw,), jnp.int32),
                       pltpu.VMEM((bpw,), jnp.int32),
                       pltpu.SemaphoreType.DMA],
    )
    def k(d2t_hbm, tok_hbm, out_hbm, idx_v, gat_v, sem):
        wid = lax.axis_index("s") * nc + lax.axis_index("c")
        base = wid * bpw
        pltpu.sync_copy(tok_hbm.at[pl.ds(base, bpw)], idx_v)
        pltpu.async_copy(d2t_hbm.at[idx_v], gat_v, sem).wait()
        for c in range(bpw // 16):
            sl = pl.ds(c * 16, 16)
            gat_v[sl] = gat_v[sl] + idx_v[sl]
        pltpu.sync_copy(gat_v, out_hbm.at[pl.ds(base, bpw)])

    return k(d2t, tokens)


def kernel(logits, d2t, max_top_k):
    tok_f, val_f, bad = _topk_fast(logits)
    tok2d, sc2d = lax.cond(
        jnp.max(bad) > 0,
        lambda: _topk_exact(logits),
        lambda: (tok_f, val_f))
    tokens = tok2d.reshape(-1) + (max_top_k - _K)
    tokens = _d2t_adjust(d2t, tokens)
    return tokens, sc2d.reshape(-1)


# speedup vs baseline: 1.1858x; 1.1858x over previous
import functools

import jax
import jax.numpy as jnp
from jax import lax
from jax.experimental import pallas as pl
from jax.experimental.pallas import tpu as pltpu
from jax.experimental.pallas import tpu_sc as plsc

_K = 8
_RB = 16
_CW = 12800
_NCH = 8
_SW = 1280
_BIG = 2**31 - 1
_NEG = float("-inf")


def _fast_body(x_ref, tok_ref, val_ref, bad_ref):
    x = x_ref[...]
    r, v = x.shape
    ns = _CW // _SW

    pos0 = lax.broadcasted_iota(jnp.int32, (r, _SW), 1)
    v1 = jnp.full((r, _SW), _NEG, jnp.float32)
    j1 = jnp.zeros((r, _SW), jnp.int32)
    v2 = jnp.full((r, _SW), _NEG, jnp.float32)
    j2 = jnp.zeros((r, _SW), jnp.int32)
    v3 = jnp.full((r, _SW), _NEG, jnp.float32)
    w3s = []
    for s in range(ns):
        w1 = x[:, s * _SW:(s + 1) * _SW]
        c1 = jnp.zeros((r, _SW), jnp.int32)
        w2 = jnp.full((r, _SW), _NEG, jnp.float32)
        c2 = jnp.zeros((r, _SW), jnp.int32)
        w3 = jnp.full((r, _SW), _NEG, jnp.float32)
        for c in range(1, _NCH):
            off = c * _CW + s * _SW
            if off >= v:
                continue
            wdt = min(_SW, v - off)
            xc = x[:, off:off + wdt]
            if wdt < _SW:
                xc = jnp.concatenate(
                    [xc, jnp.full((r, _SW - wdt), _NEG, jnp.float32)],
                    axis=1)
            gt1 = xc > w1
            gt2 = xc > w2
            gt3 = xc > w3
            cc = jnp.int32(c)
            w3 = jnp.where(gt2, w2, jnp.where(gt3, xc, w3))
            w2 = jnp.where(gt1, w1, jnp.where(gt2, xc, w2))
            c2 = jnp.where(gt1, c1, jnp.where(gt2, cc, c2))
            w1 = jnp.where(gt1, xc, w1)
            c1 = jnp.where(gt1, cc, c1)
        w3s.append(w3)
        base = s * _SW
        for (wa, ca) in ((w1, c1), (w2, c2)):
            ia = ca * _CW + (base + pos0)
            gt1 = wa > v1
            gt2 = wa > v2
            gt3 = wa > v3
            v3 = jnp.where(gt2, v2, jnp.where(gt3, wa, v3))
            v2 = jnp.where(gt1, v1, jnp.where(gt2, wa, v2))
            j2 = jnp.where(gt1, j1, jnp.where(gt2, ia, j2))
            v1 = jnp.where(gt1, wa, v1)
            j1 = jnp.where(gt1, ia, j1)

    m3 = v3
    for w3 in w3s:
        m3 = jnp.maximum(m3, w3)
    cand = jnp.concatenate([v1, v2], axis=1)
    cidx = jnp.concatenate([j1, j2], axis=1)

    toks, vals = [], []
    for j in range(_K):
        mj = jnp.max(cand, axis=-1, keepdims=True)
        ij = jnp.min(jnp.where(cand == mj, cidx, _BIG),
                     axis=-1, keepdims=True)
        toks.append(ij)
        vals.append(mj)
        if j + 1 < _K:
            cand = jnp.where(cidx == ij, _NEG, cand)

    m0 = vals[0]
    se = jnp.sum(jnp.exp(x - m0), axis=-1, keepdims=True)
    lse = m0 + jnp.log(se)

    v7 = vals[_K - 1]
    bad_l1 = jnp.max(m3, axis=-1, keepdims=True) >= v7
    nbad = jnp.sum(bad_l1.astype(jnp.int32))

    tok_ref[...] = jnp.concatenate(toks, axis=1)
    val_ref[...] = jnp.concatenate(vals, axis=1) - lse
    bad_ref[...] = jnp.full(bad_ref.shape, nbad, jnp.int32)


def _exact_body(x_ref, tok_ref, val_ref):
    x = x_ref[...]
    r, v = x.shape
    iota = lax.broadcasted_iota(jnp.int32, (r, v), 1)
    m0 = jnp.max(x, axis=-1, keepdims=True)
    lse = m0 + jnp.log(jnp.sum(jnp.exp(x - m0), axis=-1, keepdims=True))
    work = x
    toks, vals = [], []
    for j in range(_K):
        mj = m0 if j == 0 else jnp.max(work, axis=-1, keepdims=True)
        ij = jnp.min(jnp.where(work == mj, iota, _BIG),
                     axis=-1, keepdims=True)
        toks.append(ij)
        vals.append(mj)
        if j + 1 < _K:
            work = jnp.where(iota == ij, _NEG, work)
    tok_ref[...] = jnp.concatenate(toks, axis=1)
    val_ref[...] = jnp.concatenate(vals, axis=1) - lse


def _topk_fast(logits):
    b, v = logits.shape
    return pl.pallas_call(
        _fast_body,
        grid=(b // _RB,),
        in_specs=[pl.BlockSpec((_RB, v), lambda i: (i, 0))],
        out_specs=[pl.BlockSpec((_RB, _K), lambda i: (i, 0)),
                   pl.BlockSpec((_RB, _K), lambda i: (i, 0)),
                   pl.BlockSpec((_RB, _K), lambda i: (i, 0))],
        out_shape=[jax.ShapeDtypeStruct((b, _K), jnp.int32),
                   jax.ShapeDtypeStruct((b, _K), jnp.float32),
                   jax.ShapeDtypeStruct((b, _K), jnp.int32)],
        compiler_params=pltpu.CompilerParams(
            dimension_semantics=("arbitrary",)),
    )(logits)


def _topk_exact(logits):
    b, v = logits.shape
    rb = 8
    return pl.pallas_call(
        _exact_body,
        grid=(b // rb,),
        in_specs=[pl.BlockSpec((rb, v), lambda i: (i, 0))],
        out_specs=[pl.BlockSpec((rb, _K), lambda i: (i, 0)),
                   pl.BlockSpec((rb, _K), lambda i: (i, 0))],
        out_shape=[jax.ShapeDtypeStruct((b, _K), jnp.int32),
                   jax.ShapeDtypeStruct((b, _K), jnp.float32)],
    )(logits)


def _d2t_adjust(d2t, tokens):
    info = plsc.get_sparse_core_info()
    nc, ns = info.num_cores, info.num_subcores
    nw = nc * ns
    b = tokens.shape[0]
    bpw = b // nw
    mesh = plsc.VectorSubcoreMesh(core_axis_name="c", subcore_axis_name="s")

    @functools.partial(
        pl.kernel, mesh=mesh,
        out_type=jax.ShapeDtypeStruct((b,), jnp.int32),
        scratch_types=[pltpu.VMEM((bpw,), jnp.int32),
                       pltpu.VMEM((bpw,), jnp.int32),
                       pltpu.SemaphoreType.DMA],
    )
    def k(d2t_hbm, tok_hbm, out_hbm, idx_v, gat_v, sem):
        wid = lax.axis_index("s") * nc + lax.axis_index("c")
        base = wid * bpw
        pltpu.sync_copy(tok_hbm.at[pl.ds(base, bpw)], idx_v)
        pltpu.async_copy(d2t_hbm.at[idx_v], gat_v, sem).wait()
        for c in range(bpw // 16):
            sl = pl.ds(c * 16, 16)
            gat_v[sl] = gat_v[sl] + idx_v[sl]
        pltpu.sync_copy(gat_v, out_hbm.at[pl.ds(base, bpw)])

    return k(d2t, tokens)


def kernel(logits, d2t, max_top_k):
    tok_f, val_f, bad = _topk_fast(logits)
    tok2d, sc2d = lax.cond(
        jnp.max(bad) > 0,
        lambda: _topk_exact(logits),
        lambda: (tok_f, val_f))
    tokens = tok2d.reshape(-1) + (max_top_k - _K)
    tokens = _d2t_adjust(d2t, tokens)
    return tokens, sc2d.reshape(-1)


# speedup vs baseline: 1.2580x; 1.0609x over previous
import functools

import jax
import jax.numpy as jnp
from jax import lax
from jax.experimental import pallas as pl
from jax.experimental.pallas import tpu as pltpu
from jax.experimental.pallas import tpu_sc as plsc

_K = 8
_RB = 16
_CW = 12800
_NCH = 8
_SW = 1280
_BIG = 2**31 - 1
_NEG = float("-inf")


def _fast_body(x_ref, tok_ref, val_ref, bad_ref):
    r, v = x_ref.shape
    ns = _CW // _SW

    pos0 = lax.broadcasted_iota(jnp.int32, (r, _SW), 1)
    v1 = jnp.full((r, _SW), _NEG, jnp.float32)
    j1 = jnp.zeros((r, _SW), jnp.int32)
    v2 = jnp.full((r, _SW), _NEG, jnp.float32)
    j2 = jnp.zeros((r, _SW), jnp.int32)
    v3 = jnp.full((r, _SW), _NEG, jnp.float32)
    w3s = []
    for s in range(ns):
        w1 = x_ref[:, s * _SW:(s + 1) * _SW]
        c1 = jnp.zeros((r, _SW), jnp.int32)
        w2 = jnp.full((r, _SW), _NEG, jnp.float32)
        c2 = jnp.zeros((r, _SW), jnp.int32)
        w3 = jnp.full((r, _SW), _NEG, jnp.float32)
        for c in range(1, _NCH):
            off = c * _CW + s * _SW
            if off >= v:
                continue
            wdt = min(_SW, v - off)
            xc = x_ref[:, off:off + wdt]
            if wdt < _SW:
                xc = jnp.concatenate(
                    [xc, jnp.full((r, _SW - wdt), _NEG, jnp.float32)],
                    axis=1)
            gt1 = xc > w1
            gt2 = xc > w2
            gt3 = xc > w3
            cc = jnp.int32(c)
            w3 = jnp.where(gt2, w2, jnp.where(gt3, xc, w3))
            w2 = jnp.where(gt1, w1, jnp.where(gt2, xc, w2))
            c2 = jnp.where(gt1, c1, jnp.where(gt2, cc, c2))
            w1 = jnp.where(gt1, xc, w1)
            c1 = jnp.where(gt1, cc, c1)
        w3s.append(w3)
        base = s * _SW
        for (wa, ca) in ((w1, c1), (w2, c2)):
            ia = ca * _CW + (base + pos0)
            gt1 = wa > v1
            gt2 = wa > v2
            gt3 = wa > v3
            v3 = jnp.where(gt2, v2, jnp.where(gt3, wa, v3))
            v2 = jnp.where(gt1, v1, jnp.where(gt2, wa, v2))
            j2 = jnp.where(gt1, j1, jnp.where(gt2, ia, j2))
            v1 = jnp.where(gt1, wa, v1)
            j1 = jnp.where(gt1, ia, j1)

    m3 = v3
    for w3 in w3s:
        m3 = jnp.maximum(m3, w3)
    cand = jnp.concatenate([v1, v2], axis=1)
    cidx = jnp.concatenate([j1, j2], axis=1)

    toks, vals = [], []
    for j in range(_K):
        mj = jnp.max(cand, axis=-1, keepdims=True)
        ij = jnp.min(jnp.where(cand == mj, cidx, _BIG),
                     axis=-1, keepdims=True)
        toks.append(ij)
        vals.append(mj)
        if j + 1 < _K:
            cand = jnp.where(cidx == ij, _NEG, cand)

    m0 = vals[0]
    separt = jnp.zeros((r, _SW), jnp.float32)
    for off in range(0, v, _SW):
        wdt = min(_SW, v - off)
        xc = x_ref[:, off:off + wdt]
        if wdt < _SW:
            xc = jnp.concatenate(
                [xc, jnp.full((r, _SW - wdt), _NEG, jnp.float32)], axis=1)
        separt = separt + jnp.exp(xc - m0)
    se = jnp.sum(separt, axis=-1, keepdims=True)
    lse = m0 + jnp.log(se)

    v7 = vals[_K - 1]
    bad_l1 = jnp.max(m3, axis=-1, keepdims=True) >= v7
    nbad = jnp.sum(bad_l1.astype(jnp.int32))

    tok_ref[...] = jnp.concatenate(toks, axis=1)
    val_ref[...] = jnp.concatenate(vals, axis=1) - lse
    bad_ref[...] = jnp.full(bad_ref.shape, nbad, jnp.int32)


def _exact_body(x_ref, tok_ref, val_ref):
    x = x_ref[...]
    r, v = x.shape
    iota = lax.broadcasted_iota(jnp.int32, (r, v), 1)
    m0 = jnp.max(x, axis=-1, keepdims=True)
    lse = m0 + jnp.log(jnp.sum(jnp.exp(x - m0), axis=-1, keepdims=True))
    work = x
    toks, vals = [], []
    for j in range(_K):
        mj = m0 if j == 0 else jnp.max(work, axis=-1, keepdims=True)
        ij = jnp.min(jnp.where(work == mj, iota, _BIG),
                     axis=-1, keepdims=True)
        toks.append(ij)
        vals.append(mj)
        if j + 1 < _K:
            work = jnp.where(iota == ij, _NEG, work)
    tok_ref[...] = jnp.concatenate(toks, axis=1)
    val_ref[...] = jnp.concatenate(vals, axis=1) - lse


def _topk_fast(logits):
    b, v = logits.shape
    return pl.pallas_call(
        _fast_body,
        grid=(b // _RB,),
        in_specs=[pl.BlockSpec((_RB, v), lambda i: (i, 0))],
        out_specs=[pl.BlockSpec((_RB, _K), lambda i: (i, 0)),
                   pl.BlockSpec((_RB, _K), lambda i: (i, 0)),
                   pl.BlockSpec((_RB, _K), lambda i: (i, 0))],
        out_shape=[jax.ShapeDtypeStruct((b, _K), jnp.int32),
                   jax.ShapeDtypeStruct((b, _K), jnp.float32),
                   jax.ShapeDtypeStruct((b, _K), jnp.int32)],
        compiler_params=pltpu.CompilerParams(
            dimension_semantics=("arbitrary",)),
    )(logits)


def _topk_exact(logits):
    b, v = logits.shape
    rb = 8
    return pl.pallas_call(
        _exact_body,
        grid=(b // rb,),
        in_specs=[pl.BlockSpec((rb, v), lambda i: (i, 0))],
        out_specs=[pl.BlockSpec((rb, _K), lambda i: (i, 0)),
                   pl.BlockSpec((rb, _K), lambda i: (i, 0))],
        out_shape=[jax.ShapeDtypeStruct((b, _K), jnp.int32),
                   jax.ShapeDtypeStruct((b, _K), jnp.float32)],
    )(logits)


def _d2t_adjust(d2t, tokens):
    info = plsc.get_sparse_core_info()
    nc, ns = info.num_cores, info.num_subcores
    nw = nc * ns
    b = tokens.shape[0]
    bpw = b // nw
    mesh = plsc.VectorSubcoreMesh(core_axis_name="c", subcore_axis_name="s")

    @functools.partial(
        pl.kernel, mesh=mesh,
        out_type=jax.ShapeDtypeStruct((b,), jnp.int32),
        scratch_types=[pltpu.VMEM((bpw,), jnp.int32),
                       pltpu.VMEM((bpw,), jnp.int32),
                       pltpu.SemaphoreType.DMA],
    )
    def k(d2t_hbm, tok_hbm, out_hbm, idx_v, gat_v, sem):
        wid = lax.axis_index("s") * nc + lax.axis_index("c")
        base = wid * bpw
        pltpu.sync_copy(tok_hbm.at[pl.ds(base, bpw)], idx_v)
        pltpu.async_copy(d2t_hbm.at[idx_v], gat_v, sem).wait()
        for c in range(bpw // 16):
            sl = pl.ds(c * 16, 16)
            gat_v[sl] = gat_v[sl] + idx_v[sl]
        pltpu.sync_copy(gat_v, out_hbm.at[pl.ds(base, bpw)])

    return k(d2t, tokens)


def kernel(logits, d2t, max_top_k):
    tok_f, val_f, bad = _topk_fast(logits)
    tok2d, sc2d = lax.cond(
        jnp.max(bad) > 0,
        lambda: _topk_exact(logits),
        lambda: (tok_f, val_f))
    tokens = tok2d.reshape(-1) + (max_top_k - _K)
    tokens = _d2t_adjust(d2t, tokens)
    return tokens, sc2d.reshape(-1)
